# baseline (speedup 1.0000x reference)
import functools

import jax
import jax.numpy as jnp
from jax import lax
from jax.experimental import pallas as pl
from jax.experimental.pallas import tpu as pltpu
from jax.experimental.pallas import tpu_sc as plsc

_REPS = (2, 2, 2, 2, 2, 3, 2, 2, 2, 2)
_STARTS = (0, 2, 4, 6, 8, 10, 13, 15, 17, 19)
_NIN = 10
_NOUT = 21
_NF = 20
_NBUF = 5
_PREF = 3


def _make(slab_rows):
    info = plsc.get_sparse_core_info()
    nc, ns = info.num_cores, info.num_subcores
    nw = nc * ns
    piece = slab_rows // nw
    assert piece * nw == slab_rows and piece % 8 == 0
    n_items = _NF * _NIN

    mesh = plsc.VectorSubcoreMesh(core_axis_name="c", subcore_axis_name="s")

    @functools.partial(
        pl.kernel, mesh=mesh,
        out_type=jax.ShapeDtypeStruct((_NF * _NOUT * slab_rows, 128),
                                      jnp.float32),
        scratch_types=(
            [pltpu.VMEM((_NBUF * piece, 128), jnp.float32)]
            + [pltpu.SemaphoreType.DMA] * (2 * _NBUF)
        ),
    )
    def k(in_hbm, out_hbm, buf, *sems):
        rsem = sems[:_NBUF]
        wsem = sems[_NBUF:]
        wid = lax.axis_index("s") * nc + lax.axis_index("c")
        poff = wid * piece

        def read_start(item, b):
            pltpu.async_copy(
                in_hbm.at[pl.ds(item * slab_rows + poff, piece)],
                buf.at[pl.ds(b * piece, piece)], rsem[b])

        def read_wait(item, b):
            pltpu.make_async_copy(
                in_hbm.at[pl.ds(item * slab_rows + poff, piece)],
                buf.at[pl.ds(b * piece, piece)], rsem[b]).wait()

        def write_start(orow, b):
            pltpu.async_copy(
                buf.at[pl.ds(b * piece, piece)],
                out_hbm.at[pl.ds(orow * slab_rows + poff, piece)], wsem[b])

        def write_wait(orow, b):
            pltpu.make_async_copy(
                buf.at[pl.ds(b * piece, piece)],
                out_hbm.at[pl.ds(orow * slab_rows + poff, piece)],
                wsem[b]).wait()

        for kk in range(_PREF):
            read_start(kk, kk % _NBUF)

        def fbody(f, _):
            item0 = f * _NIN
            for i in range(_NIN):
                b = i % _NBUF
                item = item0 + i
                read_wait(item, b)
                orow0 = f * _NOUT + _STARTS[i]
                for r in range(_REPS[i]):
                    write_start(orow0 + r, b)

                bd = (i + _PREF) % _NBUF
                item_d = item + _PREF
                prev = item_d - _NBUF
                ip = (i + _PREF) % _NIN

                @pl.when(item_d < n_items)
                def _prefetch():
                    @pl.when(prev >= 0)
                    def _drain_prev():
                        pi = (i + _PREF - _NBUF) % _NIN
                        prow0 = (prev // _NIN) * _NOUT + _STARTS[pi]
                        for r in range(_REPS[pi]):
                            write_wait(prow0 + r, bd)

                    read_start(item_d, bd)
            return _

        lax.fori_loop(0, _NF, fbody, None)

        for i in range(_NIN - _NBUF, _NIN):
            b = i % _NBUF
            prow0 = (_NF - 1) * _NOUT + _STARTS[i]
            for r in range(_REPS[i]):
                write_wait(prow0 + r, b)

    return k


def kernel(data2):
    B, F, N, D = data2.shape
    slab_rows = D * B // 128
    x = jnp.transpose(data2, (1, 2, 3, 0)).reshape(F * N * slab_rows, 128)
    out = _make(slab_rows)(x)
    out = out.reshape(F, _NOUT, D, B)
    return jnp.transpose(out, (3, 0, 1, 2))
